# baseline (speedup 1.0000x reference)
import functools

import jax
import jax.numpy as jnp
from jax import lax
from jax.experimental import pallas as pl
from jax.experimental.pallas import tpu as pltpu
from jax.experimental.pallas import tpu_sc as plsc

T = 2048
E = 64
H = 768
I = 1024
P = 2560
CHUNK = 64
BI = 512
W = 64
NC = 2
NSUB = 16
NW = NC * NSUB


def _router_body(l_ref, dest_ref, gate_ref, meta_ref):
    l = l_ref[...]
    m = jnp.max(l, axis=1, keepdims=True)
    s = jnp.sum(jnp.exp(l - m), axis=1, keepdims=True)
    gate_ref[...] = jnp.broadcast_to(1.0 / s, (T, 128))

    oh = (l == m).astype(jnp.float32)
    tri_e = (lax.broadcasted_iota(jnp.int32, (E, E), 0)
             <= lax.broadcasted_iota(jnp.int32, (E, E), 1)).astype(jnp.float32)
    ecum = jnp.dot(oh, tri_e, preferred_element_type=jnp.float32)
    oh = oh * (ecum == 1.0).astype(jnp.float32)

    cnt = jnp.sum(oh, axis=0, keepdims=True)
    cnt_i = cnt.astype(jnp.int32)
    cnt8 = ((cnt_i + 7) & ~7).astype(jnp.float32)
    stri_e = (lax.broadcasted_iota(jnp.int32, (E, E), 0)
              < lax.broadcasted_iota(jnp.int32, (E, E), 1)).astype(jnp.float32)
    offs = jnp.dot(cnt8, stri_e, preferred_element_type=jnp.float32,
                   precision=lax.Precision.HIGHEST)

    B = 256
    tri_b = (lax.broadcasted_iota(jnp.int32, (B, B), 1)
             <= lax.broadcasted_iota(jnp.int32, (B, B), 0)).astype(jnp.float32)

    carry = jnp.zeros((1, E), jnp.float32)
    for b in range(T // B):
        r0 = b * B
        ohb = oh[r0:r0 + B, :]
        cumb = jnp.dot(tri_b, ohb, preferred_element_type=jnp.float32) + carry
        destb = jnp.sum(ohb * (offs + cumb), axis=1, keepdims=True) - 1.0
        dest_ref[r0:r0 + B, :] = destb.astype(jnp.int32)
        carry = carry + jnp.sum(ohb, axis=0, keepdims=True)

    row = jnp.concatenate([offs.astype(jnp.int32), cnt_i], axis=1)
    meta_ref[...] = jnp.broadcast_to(row, (8, 128))


def _router(router_logits):
    return pl.pallas_call(
        _router_body,
        out_shape=(
            jax.ShapeDtypeStruct((T, 1), jnp.int32),
            jax.ShapeDtypeStruct((T, 128), jnp.float32),
            jax.ShapeDtypeStruct((8, 128), jnp.int32),
        ),
    )(router_logits)


@functools.lru_cache(maxsize=None)
def _sc_kernels():
    mesh = plsc.VectorSubcoreMesh(
        core_axis_name="core", subcore_axis_name="subcore")
    bpw = T // NW

    KCH = 4
    CH = bpw // KCH

    @functools.partial(
        pl.kernel,
        out_type=(jax.ShapeDtypeStruct((P, H), jnp.float32),
                  jax.ShapeDtypeStruct((P, 128), jnp.float32)),
        mesh=mesh,
        scratch_types=[pltpu.VMEM((KCH, CH), jnp.int32),
                       pltpu.VMEM((bpw, H), jnp.float32),
                       pltpu.VMEM((bpw, 128), jnp.float32)]
        + [pltpu.SemaphoreType.DMA] * (KCH + 2),
    )
    def dispatch(x_hbm, g_hbm, i_hbm, o_hbm, og_hbm,
                 idx_v, rows_v, g_v, *sems):
        wid = lax.axis_index("subcore") * NC + lax.axis_index("core")
        base = wid * bpw
        pltpu.sync_copy(i_hbm.at[wid], idx_v)
        loads = [
            pltpu.async_copy(x_hbm.at[pl.ds(base + k * CH, CH)],
                             rows_v.at[pl.ds(k * CH, CH)], sems[k])
            for k in range(KCH)
        ]
        gload = pltpu.async_copy(g_hbm.at[pl.ds(base, bpw)], g_v, sems[KCH])
        stores = []
        for k in range(KCH):
            loads[k].wait()
            stores.append(pltpu.async_copy(
                rows_v.at[pl.ds(k * CH, CH)], o_hbm.at[idx_v.at[k]],
                sems[KCH + 1]))
        gload.wait()
        for k in range(KCH):
            stores.append(pltpu.async_copy(
                g_v.at[pl.ds(k * CH, CH)], og_hbm.at[idx_v.at[k]],
                sems[KCH + 1]))
        for s in stores:
            s.wait()

    @functools.partial(
        pl.kernel,
        out_type=jax.ShapeDtypeStruct((T, H), jnp.float32),
        mesh=mesh,
        scratch_types=[pltpu.VMEM((KCH, CH), jnp.int32),
                       pltpu.VMEM((bpw, H), jnp.float32)]
        + [pltpu.SemaphoreType.DMA] * (KCH + 1),
    )
    def combine(y_hbm, i_hbm, o_hbm, idx_v, rows_v, *sems):
        wid = lax.axis_index("subcore") * NC + lax.axis_index("core")
        base = wid * bpw
        pltpu.sync_copy(i_hbm.at[wid], idx_v)
        gathers = [
            pltpu.async_copy(y_hbm.at[idx_v.at[k]],
                             rows_v.at[pl.ds(k * CH, CH)], sems[k])
            for k in range(KCH)
        ]
        stores = []
        for k in range(KCH):
            gathers[k].wait()
            stores.append(pltpu.async_copy(
                rows_v.at[pl.ds(k * CH, CH)],
                o_hbm.at[pl.ds(base + k * CH, CH)], sems[KCH]))
        for s in stores:
            s.wait()

    return dispatch, combine


EPG = 1


def _ffn_body(meta_ref, x_ref, g_ref, w1_ref, w3_ref, w2_ref, y_ref):
    step = pl.program_id(0)

    for sub in range(EPG):
        e = step * EPG + sub
        off = meta_ref[e]
        cnt = meta_ref[E + e]
        w1 = w1_ref[sub]
        w3 = w3_ref[sub]
        w2 = w2_ref[sub]
        nch = (cnt + CHUNK - 1) // CHUNK

        def body(c, carry):
            start = pl.multiple_of(off + c * CHUNK, 8)
            xg = x_ref[pl.ds(start, CHUNK), :]
            a = jnp.dot(xg, w1, preferred_element_type=jnp.float32,
                        precision=lax.Precision.DEFAULT)
            b = jnp.dot(xg, w3, preferred_element_type=jnp.float32,
                        precision=lax.Precision.DEFAULT)
            g = g_ref[pl.ds(start, CHUNK), 0:1]
            h = a * (1.0 / (1.0 + jnp.exp(-a))) * b
            rid = lax.broadcasted_iota(jnp.int32, (CHUNK, I), 0) + c * CHUNK
            h = jnp.where(rid < cnt, h, 0.0)
            y_ref[pl.ds(start, CHUNK), :] = jnp.dot(
                h, w2, preferred_element_type=jnp.float32,
                precision=lax.Precision.DEFAULT) * g
            return carry

        lax.fori_loop(0, nch, body, 0)


def _ffn(meta_vec, x_sorted, gate_sorted, w1, w3, w2):
    return pl.pallas_call(
        _ffn_body,
        grid=(E // EPG,),
        in_specs=[
            pl.BlockSpec(memory_space=pltpu.SMEM),
            pl.BlockSpec((P, H), lambda e: (0, 0)),
            pl.BlockSpec((P, 128), lambda e: (0, 0)),
            pl.BlockSpec((EPG, H, I), lambda e: (e, 0, 0)),
            pl.BlockSpec((EPG, H, I), lambda e: (e, 0, 0)),
            pl.BlockSpec((EPG, I, H), lambda e: (e, 0, 0)),
        ],
        out_specs=pl.BlockSpec((P, H), lambda e: (0, 0)),
        out_shape=jax.ShapeDtypeStruct((P, H), jnp.float32),
        compiler_params=pltpu.CompilerParams(
            dimension_semantics=("arbitrary",),
            vmem_limit_bytes=64 * 1024 * 1024),
    )(meta_vec, x_sorted, gate_sorted, w1, w3, w2)


def kernel(hidden_states, router_logits, w1, w2, w3):
    dispatch, combine = _sc_kernels()
    dest, gate, meta = _router(router_logits)
    dest_row = dest.reshape(NW, 4, T // NW // 4)
    meta_vec = meta[0]
    x_sorted, gate_sorted = dispatch(hidden_states, gate, dest_row)
    y_sorted = _ffn(meta_vec, x_sorted, gate_sorted, w1, w3, w2)
    return combine(y_sorted, dest_row)


# speedup vs baseline: 1.0017x; 1.0017x over previous
import functools

import jax
import jax.numpy as jnp
from jax import lax
from jax.experimental import pallas as pl
from jax.experimental.pallas import tpu as pltpu
from jax.experimental.pallas import tpu_sc as plsc

T = 2048
E = 64
H = 768
I = 1024
P = 2560
CHUNK = 64
NC = 2
NSUB = 16
NW = NC * NSUB


def _router_body(l_ref, dest_ref, gate_ref, meta_ref):
    l = l_ref[...]
    m = jnp.max(l, axis=1, keepdims=True)
    s = jnp.sum(jnp.exp(l - m), axis=1, keepdims=True)
    gate_ref[...] = jnp.broadcast_to(1.0 / s, (T, 128))

    oh = (l == m).astype(jnp.float32)
    tri_e = (lax.broadcasted_iota(jnp.int32, (E, E), 0)
             <= lax.broadcasted_iota(jnp.int32, (E, E), 1)).astype(jnp.float32)
    ecum = jnp.dot(oh, tri_e, preferred_element_type=jnp.float32)
    oh = oh * (ecum == 1.0).astype(jnp.float32)

    cnt = jnp.sum(oh, axis=0, keepdims=True)
    cnt_i = cnt.astype(jnp.int32)
    cnt8 = ((cnt_i + 7) & ~7).astype(jnp.float32)
    stri_e = (lax.broadcasted_iota(jnp.int32, (E, E), 0)
              < lax.broadcasted_iota(jnp.int32, (E, E), 1)).astype(jnp.float32)
    offs = jnp.dot(cnt8, stri_e, preferred_element_type=jnp.float32,
                   precision=lax.Precision.HIGHEST)

    B = 256
    tri_b = (lax.broadcasted_iota(jnp.int32, (B, B), 1)
             <= lax.broadcasted_iota(jnp.int32, (B, B), 0)).astype(jnp.float32)

    carry = jnp.zeros((1, E), jnp.float32)
    for b in range(T // B):
        r0 = b * B
        ohb = oh[r0:r0 + B, :]
        cumb = jnp.dot(tri_b, ohb, preferred_element_type=jnp.float32) + carry
        destb = jnp.sum(ohb * (offs + cumb), axis=1, keepdims=True) - 1.0
        dest_ref[r0:r0 + B, :] = destb.astype(jnp.int32)
        carry = carry + jnp.sum(ohb, axis=0, keepdims=True)

    row = jnp.concatenate([offs.astype(jnp.int32), cnt_i], axis=1)
    meta_ref[...] = jnp.broadcast_to(row, (8, 128))


def _router(router_logits):
    return pl.pallas_call(
        _router_body,
        out_shape=(
            jax.ShapeDtypeStruct((T, 1), jnp.int32),
            jax.ShapeDtypeStruct((T, 128), jnp.float32),
            jax.ShapeDtypeStruct((8, 128), jnp.int32),
        ),
    )(router_logits)


@functools.lru_cache(maxsize=None)
def _sc_kernels():
    mesh = plsc.VectorSubcoreMesh(
        core_axis_name="core", subcore_axis_name="subcore")
    bpw = T // NW

    KCH = 4
    CH = bpw // KCH

    @functools.partial(
        pl.kernel,
        out_type=(jax.ShapeDtypeStruct((P, H), jnp.float32),
                  jax.ShapeDtypeStruct((P, 128), jnp.float32)),
        mesh=mesh,
        scratch_types=[pltpu.VMEM((KCH, CH), jnp.int32),
                       pltpu.VMEM((bpw, H), jnp.float32),
                       pltpu.VMEM((bpw, 128), jnp.float32)]
        + [pltpu.SemaphoreType.DMA] * (KCH + 2),
    )
    def dispatch(x_hbm, g_hbm, i_hbm, o_hbm, og_hbm,
                 idx_v, rows_v, g_v, *sems):
        wid = lax.axis_index("subcore") * NC + lax.axis_index("core")
        base = wid * bpw
        pltpu.sync_copy(i_hbm.at[wid], idx_v)
        loads = [
            pltpu.async_copy(x_hbm.at[pl.ds(base + k * CH, CH)],
                             rows_v.at[pl.ds(k * CH, CH)], sems[k])
            for k in range(KCH)
        ]
        gload = pltpu.async_copy(g_hbm.at[pl.ds(base, bpw)], g_v, sems[KCH])
        stores = []
        for k in range(KCH):
            loads[k].wait()
            stores.append(pltpu.async_copy(
                rows_v.at[pl.ds(k * CH, CH)], o_hbm.at[idx_v.at[k]],
                sems[KCH + 1]))
        gload.wait()
        for k in range(KCH):
            stores.append(pltpu.async_copy(
                g_v.at[pl.ds(k * CH, CH)], og_hbm.at[idx_v.at[k]],
                sems[KCH + 1]))
        for s in stores:
            s.wait()

    @functools.partial(
        pl.kernel,
        out_type=jax.ShapeDtypeStruct((T, H), jnp.float32),
        mesh=mesh,
        scratch_types=[pltpu.VMEM((KCH, CH), jnp.int32),
                       pltpu.VMEM((bpw, H), jnp.float32)]
        + [pltpu.SemaphoreType.DMA] * (KCH + 1),
    )
    def combine(y_hbm, i_hbm, o_hbm, idx_v, rows_v, *sems):
        wid = lax.axis_index("subcore") * NC + lax.axis_index("core")
        base = wid * bpw
        pltpu.sync_copy(i_hbm.at[wid], idx_v)
        gathers = [
            pltpu.async_copy(y_hbm.at[idx_v.at[k]],
                             rows_v.at[pl.ds(k * CH, CH)], sems[k])
            for k in range(KCH)
        ]
        stores = []
        for k in range(KCH):
            gathers[k].wait()
            stores.append(pltpu.async_copy(
                rows_v.at[pl.ds(k * CH, CH)],
                o_hbm.at[pl.ds(base + k * CH, CH)], sems[KCH]))
        for s in stores:
            s.wait()

    return dispatch, combine


EPG = 1


def _ffn_body(meta_ref, x_ref, g_ref, w1_ref, w3_ref, w2_ref, y_ref):
    step = pl.program_id(0)

    for sub in range(EPG):
        e = step * EPG + sub
        off = meta_ref[e]
        cnt = meta_ref[E + e]
        w1 = w1_ref[sub]
        w3 = w3_ref[sub]
        w2 = w2_ref[sub]
        nch = (cnt + CHUNK - 1) // CHUNK

        def body(c, carry):
            start = pl.multiple_of(off + c * CHUNK, 8)
            xg = x_ref[pl.ds(start, CHUNK), :]
            a = jnp.dot(xg, w1, preferred_element_type=jnp.float32,
                        precision=lax.Precision.DEFAULT)
            b = jnp.dot(xg, w3, preferred_element_type=jnp.float32,
                        precision=lax.Precision.DEFAULT)
            g = g_ref[pl.ds(start, CHUNK), 0:1]
            h = a * (1.0 / (1.0 + jnp.exp(-a))) * b
            rid = lax.broadcasted_iota(jnp.int32, (CHUNK, I), 0) + c * CHUNK
            h = jnp.where(rid < cnt, h, 0.0)
            y_ref[pl.ds(start, CHUNK), :] = jnp.dot(
                h, w2, preferred_element_type=jnp.float32,
                precision=lax.Precision.DEFAULT) * g
            return carry

        lax.fori_loop(0, nch, body, 0)


def _ffn(meta_vec, x_sorted, gate_sorted, w1, w3, w2):
    return pl.pallas_call(
        _ffn_body,
        grid=(E // EPG,),
        in_specs=[
            pl.BlockSpec(memory_space=pltpu.SMEM),
            pl.BlockSpec((P, H), lambda e: (0, 0)),
            pl.BlockSpec((P, 128), lambda e: (0, 0)),
            pl.BlockSpec((EPG, H, I), lambda e: (e, 0, 0)),
            pl.BlockSpec((EPG, H, I), lambda e: (e, 0, 0)),
            pl.BlockSpec((EPG, I, H), lambda e: (e, 0, 0)),
        ],
        out_specs=pl.BlockSpec((P, H), lambda e: (0, 0)),
        out_shape=jax.ShapeDtypeStruct((P, H), jnp.float32),
        compiler_params=pltpu.CompilerParams(
            dimension_semantics=("arbitrary",),
            vmem_limit_bytes=64 * 1024 * 1024),
    )(meta_vec, x_sorted, gate_sorted, w1, w3, w2)


def kernel(hidden_states, router_logits, w1, w2, w3):
    dispatch, combine = _sc_kernels()
    dest, gate, meta = _router(router_logits)
    dest_row = dest.reshape(NW, 4, T // NW // 4)
    meta_vec = meta[0]
    x_sorted, gate_sorted = dispatch(hidden_states, gate, dest_row)
    y_sorted = _ffn(meta_vec, x_sorted, gate_sorted, w1, w3, w2)
    return combine(y_sorted, dest_row)
